# Initial kernel scaffold; baseline (speedup 1.0000x reference)
import jax
import jax.numpy as jnp
from jax.experimental import pallas as pl


def kernel(atomic_numbers, positions, edge_index, edge_vectors, cell_vectors, batch_indices, embed_W, radial_W, msg_W, readout_W):
    raise NotImplementedError("write your pallas kernel here")


# speedup vs baseline: 1.6126x; 1.6126x over previous
import functools

import jax
import jax.numpy as jnp
from jax import lax
from jax.experimental import pallas as pl
from jax.experimental.pallas import tpu as pltpu
from jax.experimental.pallas import tpu_sc as plsc

_N = 10000
_E = 320000
_B = 8
_D = 128
_NB = 8
_R_MAX = 5.0
_NUM_ELEM = 100
_N_INTER = 2
_AVG_NEIGH = 12.0

_NC = 2
_NS = 16
_NW = _NC * _NS
_N_PAD = 10240
_ROWS_W = _N_PAD // _NS
_K = 80
_EPW = _E // _NW
_NCHUNK = _EPW // _K

_mesh = plsc.VectorSubcoreMesh(core_axis_name="c", subcore_axis_name="s",
                               num_cores=_NC, num_subcores=_NS)


@functools.partial(
    pl.kernel,
    out_type=jax.ShapeDtypeStruct((_N_PAD, _D), jnp.float32),
    mesh=_mesh,
    scratch_types=[
        pltpu.VMEM((_K,), jnp.int32),
        pltpu.VMEM((_K, _D), jnp.float32),
        pltpu.SemaphoreType.DMA,
    ],
)
def _sc_embed(emb_hbm, anum_hbm, out_hbm, idx_v, rows_v, sem):
    wid = lax.axis_index("s") * _NC + lax.axis_index("c")
    rows_per_w = _N_PAD // _NW

    def body(t, carry):
        base = wid * rows_per_w + t * _K
        pltpu.sync_copy(anum_hbm.at[pl.ds(base, _K)], idx_v)
        pltpu.async_copy(emb_hbm.at[idx_v], rows_v, sem).wait()
        pltpu.sync_copy(rows_v, out_hbm.at[pl.ds(base, _K)])
        return carry

    lax.fori_loop(0, rows_per_w // _K, body, 0)


@functools.partial(
    pl.kernel,
    out_type=jax.ShapeDtypeStruct((_NC * _N_PAD, _D), jnp.float32),
    mesh=_mesh,
    scratch_types=[
        pltpu.VMEM((_K,), jnp.int32),
        pltpu.VMEM((_K,), jnp.int32),
        pltpu.VMEM((_K, _D), jnp.float32),
        pltpu.VMEM((_K, _D), jnp.float32),
        pltpu.VMEM_SHARED((_N_PAD, _D), jnp.float32),
        pltpu.SemaphoreType.DMA,
    ],
)
def _sc_interact(h_hbm, src_hbm, dst_hbm, rad_hbm, zeros_hbm, out_hbm,
                 src_v, dst_v, rows_v, rad_v, agg_sh, sem):
    cid = lax.axis_index("c")
    sid = lax.axis_index("s")
    wid = sid * _NC + cid

    pltpu.sync_copy(zeros_hbm.at[pl.ds(sid * _ROWS_W, _ROWS_W)],
                    agg_sh.at[pl.ds(sid * _ROWS_W, _ROWS_W)])
    plsc.subcore_barrier()

    def row_mul(i, carry):
        for j in range(_D // 16):
            sl = pl.ds(j * 16, 16)
            rows_v[i, sl] = rows_v[i, sl] * rad_v[i, sl]
        return carry

    def chunk(c, carry):
        base = wid * _EPW + c * _K
        pltpu.sync_copy(src_hbm.at[pl.ds(base, _K)], src_v)
        pltpu.sync_copy(dst_hbm.at[pl.ds(base, _K)], dst_v)
        pltpu.async_copy(h_hbm.at[src_v], rows_v, sem).wait()
        pltpu.sync_copy(rad_hbm.at[pl.ds(base, _K)], rad_v)
        lax.fori_loop(0, _K, row_mul, 0)
        pltpu.async_copy(rows_v, agg_sh.at[dst_v], sem, add=True).wait()
        return carry

    lax.fori_loop(0, _NCHUNK, chunk, 0)

    plsc.subcore_barrier()
    pltpu.sync_copy(agg_sh.at[pl.ds(sid * _ROWS_W, _ROWS_W)],
                    out_hbm.at[pl.ds(cid * _N_PAD + sid * _ROWS_W, _ROWS_W)])


_BE = 2000
_BN = 1024


def _radial_body(ev_ref, w_ref, out_ref):
    ev = ev_ref[...]
    r = jnp.sqrt(jnp.sum(ev * ev, axis=1, keepdims=True))
    n = lax.broadcasted_iota(jnp.int32, (1, _NB), 1).astype(jnp.float32) + 1.0
    bessel = jnp.sqrt(2.0 / _R_MAX) * jnp.sin(n * jnp.pi * r / _R_MAX) / (r + 1e-8)
    x = jnp.clip(r / _R_MAX, 0.0, 1.0)
    x2 = x * x
    x4 = x2 * x2
    x5 = x4 * x
    env = 1.0 - 21.0 * x5 + 35.0 * x5 * x - 15.0 * x5 * x2
    env = env * (r < _R_MAX).astype(jnp.float32)
    feats = bessel * env
    acc = feats[:, 0:1] * w_ref[0:1, :]
    for b in range(1, _NB):
        acc = acc + feats[:, b:b + 1] * w_ref[b:b + 1, :]
    out_ref[...] = acc


def _update_body(h_ref, a0_ref, a1_ref, w_ref, out_ref):
    agg = (a0_ref[...] + a1_ref[...]) * (1.0 / _AVG_NEIGH)
    out_ref[...] = h_ref[...] + jnp.dot(
        agg, w_ref[...], preferred_element_type=jnp.float32,
        precision=lax.Precision.HIGHEST)


def _readout_body(h_ref, bid_ref, w_ref, out_ref):
    i = pl.program_id(0)
    node_e = jnp.sum(h_ref[...] * w_ref[...], axis=1, keepdims=True)
    onehot = (bid_ref[...] == lax.broadcasted_iota(jnp.int32, (1, _B), 1))
    part = jnp.sum(onehot.astype(jnp.float32) * node_e, axis=0, keepdims=True)

    @pl.when(i == 0)
    def _():
        out_ref[...] = part

    @pl.when(i > 0)
    def _():
        out_ref[...] = out_ref[...] + part


_radial_call = pl.pallas_call(
    _radial_body,
    grid=(_E // _BE,),
    in_specs=[
        pl.BlockSpec((_BE, 3), lambda i: (i, 0)),
        pl.BlockSpec((_NB, _D), lambda i: (0, 0)),
    ],
    out_specs=pl.BlockSpec((_BE, _D), lambda i: (i, 0)),
    out_shape=jax.ShapeDtypeStruct((_E, _D), jnp.float32),
)

_update_call = pl.pallas_call(
    _update_body,
    grid=(_N_PAD // _BN,),
    in_specs=[
        pl.BlockSpec((_BN, _D), lambda i: (i, 0)),
        pl.BlockSpec((_BN, _D), lambda i: (i, 0)),
        pl.BlockSpec((_BN, _D), lambda i: (_N_PAD // _BN + i, 0)),
        pl.BlockSpec((_D, _D), lambda i: (0, 0)),
    ],
    out_specs=pl.BlockSpec((_BN, _D), lambda i: (i, 0)),
    out_shape=jax.ShapeDtypeStruct((_N_PAD, _D), jnp.float32),
)

_readout_call = pl.pallas_call(
    _readout_body,
    grid=(_N_PAD // _BN,),
    in_specs=[
        pl.BlockSpec((_BN, _D), lambda i: (i, 0)),
        pl.BlockSpec((_BN, 1), lambda i: (i, 0)),
        pl.BlockSpec((1, _D), lambda i: (0, 0)),
    ],
    out_specs=pl.BlockSpec((1, _B), lambda i: (0, 0)),
    out_shape=jax.ShapeDtypeStruct((1, _B), jnp.float32),
)


def kernel(atomic_numbers, positions, edge_index, edge_vectors, cell_vectors,
           batch_indices, embed_W, radial_W, msg_W, readout_W):
    anum_pad = jnp.concatenate(
        [atomic_numbers.astype(jnp.int32),
         jnp.zeros((_N_PAD - _N,), jnp.int32)])
    bid_pad = jnp.concatenate(
        [batch_indices.astype(jnp.int32),
         jnp.full((_N_PAD - _N,), _B, jnp.int32)]).reshape(_N_PAD, 1)
    src = edge_index[0].astype(jnp.int32)
    dst = edge_index[1].astype(jnp.int32)
    zeros = jnp.zeros((_N_PAD, _D), jnp.float32)

    h = _sc_embed(embed_W, anum_pad)
    radial = _radial_call(edge_vectors, radial_W)
    for i in range(_N_INTER):
        agg2 = _sc_interact(h, src, dst, radial, zeros)
        h = _update_call(h, agg2, agg2, msg_W[i])
    energies = _readout_call(h, bid_pad, readout_W.reshape(1, _D))[0]
    forces = jnp.zeros((_N, 3), dtype=jnp.float32)
    stress = jnp.zeros((_B, 3, 3), dtype=jnp.float32)
    return (energies, forces, stress, h[:_N])


# speedup vs baseline: 2.7016x; 1.6753x over previous
import functools

import jax
import jax.numpy as jnp
from jax import lax
from jax.experimental import pallas as pl
from jax.experimental.pallas import tpu as pltpu
from jax.experimental.pallas import tpu_sc as plsc

_N = 10000
_E = 320000
_B = 8
_D = 128
_NB = 8
_R_MAX = 5.0
_NUM_ELEM = 100
_N_INTER = 2
_AVG_NEIGH = 12.0

_NC = 2
_NS = 16
_NW = _NC * _NS
_N_PAD = 10240
_ROWS_W = _N_PAD // _NS
_K = 80
_EPW = _E // _NW
_NCHUNK = _EPW // _K

_mesh = plsc.VectorSubcoreMesh(core_axis_name="c", subcore_axis_name="s",
                               num_cores=_NC, num_subcores=_NS)


@functools.partial(
    pl.kernel,
    out_type=jax.ShapeDtypeStruct((_N_PAD, _D), jnp.float32),
    mesh=_mesh,
    scratch_types=[
        pltpu.VMEM((_K,), jnp.int32),
        pltpu.VMEM((_K, _D), jnp.float32),
        pltpu.SemaphoreType.DMA,
    ],
)
def _sc_embed(emb_hbm, anum_hbm, out_hbm, idx_v, rows_v, sem):
    wid = lax.axis_index("s") * _NC + lax.axis_index("c")
    rows_per_w = _N_PAD // _NW

    def body(t, carry):
        base = wid * rows_per_w + t * _K
        pltpu.sync_copy(anum_hbm.at[pl.ds(base, _K)], idx_v)
        pltpu.async_copy(emb_hbm.at[idx_v], rows_v, sem).wait()
        pltpu.sync_copy(rows_v, out_hbm.at[pl.ds(base, _K)])
        return carry

    lax.fori_loop(0, rows_per_w // _K, body, 0)


@functools.partial(
    pl.kernel,
    out_type=jax.ShapeDtypeStruct((_NC * _N_PAD, _D), jnp.float32),
    mesh=_mesh,
    scratch_types=[
        pltpu.VMEM((_K,), jnp.int32),
        pltpu.VMEM((_K,), jnp.int32),
        pltpu.VMEM((_K, _D), jnp.float32),
        pltpu.VMEM((_K, _D), jnp.float32),
        pltpu.VMEM_SHARED((_N_PAD, _D), jnp.float32),
        pltpu.SemaphoreType.DMA,
    ],
)
def _sc_interact(h_hbm, src_hbm, dst_hbm, rad_hbm, zeros_hbm, out_hbm,
                 src_v, dst_v, rows_v, rad_v, agg_sh, sem):
    cid = lax.axis_index("c")
    sid = lax.axis_index("s")
    wid = sid * _NC + cid

    pltpu.sync_copy(zeros_hbm.at[pl.ds(sid * _ROWS_W, _ROWS_W)],
                    agg_sh.at[pl.ds(sid * _ROWS_W, _ROWS_W)])
    plsc.subcore_barrier()

    def row_mul(i, carry):
        for j in range(_D // 16):
            sl = pl.ds(j * 16, 16)
            rows_v[i, sl] = rows_v[i, sl] * rad_v[i, sl]
        return carry

    def chunk(c, carry):
        base = wid * _EPW + c * _K
        pltpu.sync_copy(src_hbm.at[pl.ds(base, _K)], src_v)
        pltpu.sync_copy(dst_hbm.at[pl.ds(base, _K)], dst_v)
        pltpu.async_copy(h_hbm.at[src_v], rows_v, sem).wait()
        pltpu.sync_copy(rad_hbm.at[pl.ds(base, _K)], rad_v)
        lax.fori_loop(0, _K, row_mul, 0)
        pltpu.async_copy(rows_v, agg_sh.at[dst_v], sem, add=True).wait()
        return carry

    lax.fori_loop(0, _NCHUNK, chunk, 0)

    plsc.subcore_barrier()
    pltpu.sync_copy(agg_sh.at[pl.ds(sid * _ROWS_W, _ROWS_W)],
                    out_hbm.at[pl.ds(cid * _N_PAD + sid * _ROWS_W, _ROWS_W)])


_BE = 2560
_BN = 1024


def _radial_body(x_ref, y_ref, z_ref, w_ref, out_ref):
    x = x_ref[0]
    y = y_ref[0]
    z = z_ref[0]
    r = jnp.sqrt(x * x + y * y + z * z)
    u = jnp.clip(r * (1.0 / _R_MAX), 0.0, 1.0)
    u2 = u * u
    u4 = u2 * u2
    u5 = u4 * u
    env = 1.0 - 21.0 * u5 + 35.0 * u5 * u - 15.0 * u5 * u2
    env = env * (r < _R_MAX).astype(jnp.float32)
    q = jnp.sqrt(2.0 / _R_MAX) * env / (r + 1e-8)
    r8 = jnp.broadcast_to(r * (jnp.pi / _R_MAX), (_NB, _BE))
    q8 = jnp.broadcast_to(q, (_NB, _BE))
    n = (lax.broadcasted_iota(jnp.int32, (_NB, _BE), 0) + 1).astype(jnp.float32)
    feats = jnp.sin(n * r8) * q8
    out_ref[...] = lax.dot_general(
        feats, w_ref[...], (((0,), (0,)), ((), ())),
        preferred_element_type=jnp.float32, precision=lax.Precision.HIGHEST)


def _update_body(h_ref, a0_ref, a1_ref, w_ref, out_ref):
    agg = (a0_ref[...] + a1_ref[...]) * (1.0 / _AVG_NEIGH)
    out_ref[...] = h_ref[...] + jnp.dot(
        agg, w_ref[...], preferred_element_type=jnp.float32,
        precision=lax.Precision.HIGHEST)


def _readout_body(h_ref, bid_ref, w_ref, out_ref):
    i = pl.program_id(0)
    node_e = jnp.sum(h_ref[...] * w_ref[...], axis=1, keepdims=True)
    onehot = (bid_ref[...] == lax.broadcasted_iota(jnp.int32, (1, _B), 1))
    part = jnp.sum(onehot.astype(jnp.float32) * node_e, axis=0, keepdims=True)

    @pl.when(i == 0)
    def _():
        out_ref[...] = part

    @pl.when(i > 0)
    def _():
        out_ref[...] = out_ref[...] + part


_radial_call = pl.pallas_call(
    _radial_body,
    grid=(_E // _BE,),
    in_specs=[
        pl.BlockSpec((1, 1, _BE), lambda i: (i, 0, 0)),
        pl.BlockSpec((1, 1, _BE), lambda i: (i, 0, 0)),
        pl.BlockSpec((1, 1, _BE), lambda i: (i, 0, 0)),
        pl.BlockSpec((_NB, _D), lambda i: (0, 0)),
    ],
    out_specs=pl.BlockSpec((_BE, _D), lambda i: (i, 0)),
    out_shape=jax.ShapeDtypeStruct((_E, _D), jnp.float32),
)

_update_call = pl.pallas_call(
    _update_body,
    grid=(_N_PAD // _BN,),
    in_specs=[
        pl.BlockSpec((_BN, _D), lambda i: (i, 0)),
        pl.BlockSpec((_BN, _D), lambda i: (i, 0)),
        pl.BlockSpec((_BN, _D), lambda i: (_N_PAD // _BN + i, 0)),
        pl.BlockSpec((_D, _D), lambda i: (0, 0)),
    ],
    out_specs=pl.BlockSpec((_BN, _D), lambda i: (i, 0)),
    out_shape=jax.ShapeDtypeStruct((_N_PAD, _D), jnp.float32),
)

_readout_call = pl.pallas_call(
    _readout_body,
    grid=(_N_PAD // _BN,),
    in_specs=[
        pl.BlockSpec((_BN, _D), lambda i: (i, 0)),
        pl.BlockSpec((_BN, 1), lambda i: (i, 0)),
        pl.BlockSpec((1, _D), lambda i: (0, 0)),
    ],
    out_specs=pl.BlockSpec((1, _B), lambda i: (0, 0)),
    out_shape=jax.ShapeDtypeStruct((1, _B), jnp.float32),
)


def kernel(atomic_numbers, positions, edge_index, edge_vectors, cell_vectors,
           batch_indices, embed_W, radial_W, msg_W, readout_W):
    anum_pad = jnp.concatenate(
        [atomic_numbers.astype(jnp.int32),
         jnp.zeros((_N_PAD - _N,), jnp.int32)])
    bid_pad = jnp.concatenate(
        [batch_indices.astype(jnp.int32),
         jnp.full((_N_PAD - _N,), _B, jnp.int32)]).reshape(_N_PAD, 1)
    src = edge_index[0].astype(jnp.int32)
    dst = edge_index[1].astype(jnp.int32)
    zeros = jnp.zeros((_N_PAD, _D), jnp.float32)

    evx = edge_vectors[:, 0].reshape(_E // _BE, 1, _BE)
    evy = edge_vectors[:, 1].reshape(_E // _BE, 1, _BE)
    evz = edge_vectors[:, 2].reshape(_E // _BE, 1, _BE)

    h = _sc_embed(embed_W, anum_pad)
    radial = _radial_call(evx, evy, evz, radial_W)
    for i in range(_N_INTER):
        agg2 = _sc_interact(h, src, dst, radial, zeros)
        h = _update_call(h, agg2, agg2, msg_W[i])
    energies = _readout_call(h, bid_pad, readout_W.reshape(1, _D))[0]
    forces = jnp.zeros((_N, 3), dtype=jnp.float32)
    stress = jnp.zeros((_B, 3, 3), dtype=jnp.float32)
    return (energies, forces, stress, h[:_N])


# speedup vs baseline: 4.8817x; 1.8070x over previous
import functools

import jax
import jax.numpy as jnp
from jax import lax
from jax.experimental import pallas as pl
from jax.experimental.pallas import tpu as pltpu
from jax.experimental.pallas import tpu_sc as plsc

_N = 10000
_E = 320000
_B = 8
_D = 128
_NB = 8
_R_MAX = 5.0
_NUM_ELEM = 100
_N_INTER = 2
_AVG_NEIGH = 12.0

_NC = 2
_NS = 16
_NW = _NC * _NS
_N_PAD = 10240
_ROWS_W = _N_PAD // _NS
_K = 40
_EPW = _E // _NW
_NCHUNK = _EPW // _K

_mesh = plsc.VectorSubcoreMesh(core_axis_name="c", subcore_axis_name="s",
                               num_cores=_NC, num_subcores=_NS)


@functools.partial(
    pl.kernel,
    out_type=jax.ShapeDtypeStruct((_N_PAD, _D), jnp.float32),
    mesh=_mesh,
    scratch_types=[
        pltpu.VMEM((_K,), jnp.int32),
        pltpu.VMEM((_K, _D), jnp.float32),
        pltpu.SemaphoreType.DMA,
    ],
)
def _sc_embed(emb_hbm, anum_hbm, out_hbm, idx_v, rows_v, sem):
    wid = lax.axis_index("s") * _NC + lax.axis_index("c")
    rows_per_w = _N_PAD // _NW

    def body(t, carry):
        base = wid * rows_per_w + t * _K
        pltpu.sync_copy(anum_hbm.at[pl.ds(base, _K)], idx_v)
        pltpu.async_copy(emb_hbm.at[idx_v], rows_v, sem).wait()
        pltpu.sync_copy(rows_v, out_hbm.at[pl.ds(base, _K)])
        return carry

    lax.fori_loop(0, rows_per_w // _K, body, 0)


_NBUF = 3


@functools.partial(
    pl.kernel,
    out_type=jax.ShapeDtypeStruct((_NC * _N_PAD, _D), jnp.float32),
    mesh=_mesh,
    scratch_types=[
        pltpu.VMEM((_NBUF, 1, _K), jnp.int32),
        pltpu.VMEM((_NBUF, 1, _K), jnp.int32),
        pltpu.VMEM((_NBUF, _K, _D), jnp.float32),
        pltpu.VMEM((_NBUF, _K, _D), jnp.float32),
        pltpu.VMEM_SHARED((_N_PAD, _D), jnp.float32),
        pltpu.SemaphoreType.DMA((_NBUF,)),
        pltpu.SemaphoreType.DMA((_NBUF,)),
        pltpu.SemaphoreType.DMA((_NBUF,)),
        pltpu.SemaphoreType.DMA((_NBUF,)),
    ],
)
def _sc_interact(h_hbm, src_hbm, dst_hbm, rad_hbm, zeros_hbm, out_hbm,
                 isrc, idst, rows_v, rad_v, agg_sh, gsem, rsem, ssem, isem):
    cid = lax.axis_index("c")
    sid = lax.axis_index("s")
    wid = sid * _NC + cid

    pltpu.sync_copy(zeros_hbm.at[pl.ds(sid * _ROWS_W, _ROWS_W)],
                    agg_sh.at[pl.ds(sid * _ROWS_W, _ROWS_W)])
    plsc.subcore_barrier()

    ebase = wid * _EPW
    cbase = wid * _NCHUNK

    def ifetch(s, b):
        pltpu.async_copy(src_hbm.at[cbase + s], isrc.at[b], isem.at[b])
        pltpu.async_copy(dst_hbm.at[cbase + s], idst.at[b], isem.at[b])

    def iwait(s, b):
        pltpu.make_async_copy(src_hbm.at[cbase + s], isrc.at[b],
                              isem.at[b]).wait()
        pltpu.make_async_copy(dst_hbm.at[cbase + s], idst.at[b],
                              isem.at[b]).wait()

    def issue(s, p):
        pltpu.async_copy(h_hbm.at[isrc.at[p, 0]], rows_v.at[p], gsem.at[p])
        pltpu.async_copy(rad_hbm.at[pl.ds(ebase + s * _K, _K)], rad_v.at[p],
                         rsem.at[p])

    def wait_in(s, p):
        pltpu.make_async_copy(h_hbm.at[isrc.at[p, 0]], rows_v.at[p],
                              gsem.at[p]).wait()
        pltpu.make_async_copy(rad_hbm.at[pl.ds(ebase + s * _K, _K)],
                              rad_v.at[p], rsem.at[p]).wait()

    def mul(p):
        def row(i, carry):
            for j in range(_D // 16):
                sl = pl.ds(j * 16, 16)
                rows_v[p, i, sl] = rows_v[p, i, sl] * rad_v[p, i, sl]
            return carry
        lax.fori_loop(0, _K, row, 0)

    def scat(p):
        pltpu.async_copy(rows_v.at[p], agg_sh.at[idst.at[p, 0]], ssem.at[p],
                         add=True)

    def wait_scat(p):
        pltpu.make_async_copy(rows_v.at[p], agg_sh.at[idst.at[p, 0]],
                              ssem.at[p]).wait()

    ifetch(0, 0)
    ifetch(1, 1)
    iwait(0, 0)
    issue(0, 0)
    iwait(1, 1)
    issue(1, 1)
    wait_in(0, 0)
    ifetch(2, 2)
    mul(0)
    scat(0)
    iwait(2, 2)
    issue(2, 2)

    def steady(k, carry):
        for j in range(3):
            p = (1 + j) % 3
            q = j % 3
            s = 1 + 3 * k + j
            wait_in(s, p)
            wait_scat(q)
            ifetch(s + 2, q)
            mul(p)
            scat(p)
            iwait(s + 2, q)
            issue(s + 2, q)
        return carry

    lax.fori_loop(0, 82, steady, 0)

    wait_in(247, 247 % 3)
    wait_scat(249 % 3)
    ifetch(249, 249 % 3)
    mul(247 % 3)
    scat(247 % 3)
    iwait(249, 249 % 3)
    issue(249, 249 % 3)
    wait_in(248, 248 % 3)
    wait_scat(247 % 3)
    mul(248 % 3)
    scat(248 % 3)
    wait_in(249, 249 % 3)
    wait_scat(248 % 3)
    mul(249 % 3)
    scat(249 % 3)
    wait_scat(249 % 3)

    plsc.subcore_barrier()
    pltpu.sync_copy(agg_sh.at[pl.ds(sid * _ROWS_W, _ROWS_W)],
                    out_hbm.at[pl.ds(cid * _N_PAD + sid * _ROWS_W, _ROWS_W)])


_BE = 2560
_BN = 1024


def _radial_body(x_ref, y_ref, z_ref, w_ref, out_ref):
    x = x_ref[0]
    y = y_ref[0]
    z = z_ref[0]
    r = jnp.sqrt(x * x + y * y + z * z)
    u = jnp.clip(r * (1.0 / _R_MAX), 0.0, 1.0)
    u2 = u * u
    u4 = u2 * u2
    u5 = u4 * u
    env = 1.0 - 21.0 * u5 + 35.0 * u5 * u - 15.0 * u5 * u2
    env = env * (r < _R_MAX).astype(jnp.float32)
    q = jnp.sqrt(2.0 / _R_MAX) * env / (r + 1e-8)
    r8 = jnp.broadcast_to(r * (jnp.pi / _R_MAX), (_NB, _BE))
    q8 = jnp.broadcast_to(q, (_NB, _BE))
    n = (lax.broadcasted_iota(jnp.int32, (_NB, _BE), 0) + 1).astype(jnp.float32)
    feats = jnp.sin(n * r8) * q8
    out_ref[...] = lax.dot_general(
        feats, w_ref[...], (((0,), (0,)), ((), ())),
        preferred_element_type=jnp.float32, precision=lax.Precision.HIGHEST)


def _update_body(h_ref, a0_ref, a1_ref, w_ref, out_ref):
    agg = (a0_ref[...] + a1_ref[...]) * (1.0 / _AVG_NEIGH)
    out_ref[...] = h_ref[...] + jnp.dot(
        agg, w_ref[...], preferred_element_type=jnp.float32,
        precision=lax.Precision.HIGHEST)


def _readout_body(h_ref, bid_ref, w_ref, out_ref):
    i = pl.program_id(0)
    node_e = jnp.sum(h_ref[...] * w_ref[...], axis=1, keepdims=True)
    onehot = (bid_ref[...] == lax.broadcasted_iota(jnp.int32, (1, _B), 1))
    part = jnp.sum(onehot.astype(jnp.float32) * node_e, axis=0, keepdims=True)

    @pl.when(i == 0)
    def _():
        out_ref[...] = part

    @pl.when(i > 0)
    def _():
        out_ref[...] = out_ref[...] + part


_radial_call = pl.pallas_call(
    _radial_body,
    grid=(_E // _BE,),
    in_specs=[
        pl.BlockSpec((1, 1, _BE), lambda i: (i, 0, 0)),
        pl.BlockSpec((1, 1, _BE), lambda i: (i, 0, 0)),
        pl.BlockSpec((1, 1, _BE), lambda i: (i, 0, 0)),
        pl.BlockSpec((_NB, _D), lambda i: (0, 0)),
    ],
    out_specs=pl.BlockSpec((_BE, _D), lambda i: (i, 0)),
    out_shape=jax.ShapeDtypeStruct((_E, _D), jnp.float32),
)

_update_call = pl.pallas_call(
    _update_body,
    grid=(_N_PAD // _BN,),
    in_specs=[
        pl.BlockSpec((_BN, _D), lambda i: (i, 0)),
        pl.BlockSpec((_BN, _D), lambda i: (i, 0)),
        pl.BlockSpec((_BN, _D), lambda i: (_N_PAD // _BN + i, 0)),
        pl.BlockSpec((_D, _D), lambda i: (0, 0)),
    ],
    out_specs=pl.BlockSpec((_BN, _D), lambda i: (i, 0)),
    out_shape=jax.ShapeDtypeStruct((_N_PAD, _D), jnp.float32),
)

_readout_call = pl.pallas_call(
    _readout_body,
    grid=(_N_PAD // _BN,),
    in_specs=[
        pl.BlockSpec((_BN, _D), lambda i: (i, 0)),
        pl.BlockSpec((_BN, 1), lambda i: (i, 0)),
        pl.BlockSpec((1, _D), lambda i: (0, 0)),
    ],
    out_specs=pl.BlockSpec((1, _B), lambda i: (0, 0)),
    out_shape=jax.ShapeDtypeStruct((1, _B), jnp.float32),
)


def kernel(atomic_numbers, positions, edge_index, edge_vectors, cell_vectors,
           batch_indices, embed_W, radial_W, msg_W, readout_W):
    anum_pad = jnp.concatenate(
        [atomic_numbers.astype(jnp.int32),
         jnp.zeros((_N_PAD - _N,), jnp.int32)])
    bid_pad = jnp.concatenate(
        [batch_indices.astype(jnp.int32),
         jnp.full((_N_PAD - _N,), _B, jnp.int32)]).reshape(_N_PAD, 1)
    src = edge_index[0].astype(jnp.int32).reshape(_E // _K, 1, _K)
    dst = edge_index[1].astype(jnp.int32).reshape(_E // _K, 1, _K)
    zeros = jnp.zeros((_N_PAD, _D), jnp.float32)

    evx = edge_vectors[:, 0].reshape(_E // _BE, 1, _BE)
    evy = edge_vectors[:, 1].reshape(_E // _BE, 1, _BE)
    evz = edge_vectors[:, 2].reshape(_E // _BE, 1, _BE)

    h = _sc_embed(embed_W, anum_pad)
    radial = _radial_call(evx, evy, evz, radial_W)
    for i in range(_N_INTER):
        agg2 = _sc_interact(h, src, dst, radial, zeros)
        h = _update_call(h, agg2, agg2, msg_W[i])
    energies = _readout_call(h, bid_pad, readout_W.reshape(1, _D))[0]
    forces = jnp.zeros((_N, 3), dtype=jnp.float32)
    stress = jnp.zeros((_B, 3, 3), dtype=jnp.float32)
    return (energies, forces, stress, h[:_N])
